# baseline (speedup 1.0000x reference)
import functools

import jax
import jax.numpy as jnp
from jax import lax
from jax.experimental import pallas as pl
from jax.experimental.pallas import tpu as pltpu
from jax.experimental.pallas import tpu_sc as plsc

NN = 10000
NPAD = 10240
RPT = NPAD // 16
NE = 320000
ETOT = NE + NN
EPT = 10752
EPAD = EPT * 32
BB = 128
NBATCH = EPT // BB
WROW = 80
DROW = 16
BLK = 1024
GRID = NPAD // BLK

_f32 = jnp.float32


def _k1_body(x_ref, w1_ref, a1s_ref, a1d_ref, ts_ref, td_ref, mx_ref):
    h = jnp.dot(x_ref[...], w1_ref[...], preferred_element_type=_f32)
    asv = jnp.dot(h, a1s_ref[...], preferred_element_type=_f32)
    adv = jnp.dot(h, a1d_ref[...], preferred_element_type=_f32)
    ts_ref[...] = jnp.concatenate([h, jnp.ones((BLK, 8), _f32), asv], axis=1)
    td_ref[...] = jnp.concatenate([adv, jnp.zeros((BLK, 8), _f32)], axis=1)
    m = jnp.max(jnp.concatenate([asv, adv], axis=1), axis=0, keepdims=True)
    mfull = jnp.concatenate(
        [jnp.broadcast_to(m, (8, 16)), jnp.zeros((8, 112), _f32)], axis=1)
    i = pl.program_id(0)

    @pl.when(i == 0)
    def _():
        mx_ref[...] = mfull

    @pl.when(i != 0)
    def _():
        mx_ref[...] = jnp.maximum(mx_ref[...], mfull)


def _stage1(xp, w1, a1s, a1d):
    return pl.pallas_call(
        _k1_body,
        grid=(GRID,),
        in_specs=[
            pl.BlockSpec((BLK, 128), lambda i: (i, 0)),
            pl.BlockSpec((128, 64), lambda i: (0, 0)),
            pl.BlockSpec((64, 8), lambda i: (0, 0)),
            pl.BlockSpec((64, 8), lambda i: (0, 0)),
        ],
        out_specs=[
            pl.BlockSpec((BLK, WROW), lambda i: (i, 0)),
            pl.BlockSpec((BLK, DROW), lambda i: (i, 0)),
            pl.BlockSpec((8, 128), lambda i: (0, 0)),
        ],
        out_shape=[
            jax.ShapeDtypeStruct((NPAD, WROW), _f32),
            jax.ShapeDtypeStruct((NPAD, DROW), _f32),
            jax.ShapeDtypeStruct((8, 128), _f32),
        ],
    )(xp, w1, a1s, a1d)


def _k2_body(p0_ref, p1_ref, b1_ref, w2_ref, a2s_ref, a2d_ref,
             emb_ref, ts_ref, td_ref, mx_ref):
    acc = p0_ref[...] + p1_ref[...]
    numer = acc[:, 0:64]
    denom = acc[:, 64:72]
    r8 = (lax.broadcasted_iota(jnp.int32, (8, 64), 1) // 8 ==
          lax.broadcasted_iota(jnp.int32, (8, 64), 0)).astype(_f32)
    db = jnp.dot(denom, r8, preferred_element_type=_f32)
    db = jnp.where(db > 0, db, 1.0)
    out1 = numer / db + b1_ref[...]
    emb_ref[...] = out1
    hact = jnp.where(out1 > 0, out1, jnp.exp(jnp.minimum(out1, 0.0)) - 1.0)
    g = jnp.dot(hact, w2_ref[...], preferred_element_type=_f32)
    asv = jnp.dot(g, a2s_ref[...], preferred_element_type=_f32)
    adv = jnp.dot(g, a2d_ref[...], preferred_element_type=_f32)
    ts_ref[...] = jnp.concatenate([g, jnp.ones((BLK, 8), _f32), asv], axis=1)
    td_ref[...] = jnp.concatenate([adv, jnp.zeros((BLK, 8), _f32)], axis=1)
    m = jnp.max(jnp.concatenate([asv, adv], axis=1), axis=0, keepdims=True)
    mfull = jnp.concatenate(
        [jnp.broadcast_to(m, (8, 16)), jnp.zeros((8, 112), _f32)], axis=1)
    i = pl.program_id(0)

    @pl.when(i == 0)
    def _():
        mx_ref[...] = mfull

    @pl.when(i != 0)
    def _():
        mx_ref[...] = jnp.maximum(mx_ref[...], mfull)


def _stage2(p0, p1, b1, w2, a2s, a2d):
    return pl.pallas_call(
        _k2_body,
        grid=(GRID,),
        in_specs=[
            pl.BlockSpec((BLK, WROW), lambda i: (i, 0)),
            pl.BlockSpec((BLK, WROW), lambda i: (i, 0)),
            pl.BlockSpec((1, 64), lambda i: (0, 0)),
            pl.BlockSpec((64, 64), lambda i: (0, 0)),
            pl.BlockSpec((64, 8), lambda i: (0, 0)),
            pl.BlockSpec((64, 8), lambda i: (0, 0)),
        ],
        out_specs=[
            pl.BlockSpec((BLK, 64), lambda i: (i, 0)),
            pl.BlockSpec((BLK, WROW), lambda i: (i, 0)),
            pl.BlockSpec((BLK, DROW), lambda i: (i, 0)),
            pl.BlockSpec((8, 128), lambda i: (0, 0)),
        ],
        out_shape=[
            jax.ShapeDtypeStruct((NPAD, 64), _f32),
            jax.ShapeDtypeStruct((NPAD, WROW), _f32),
            jax.ShapeDtypeStruct((NPAD, DROW), _f32),
            jax.ShapeDtypeStruct((8, 128), _f32),
        ],
    )(p0, p1, b1, w2, a2s, a2d)


def _k3_body(p0_ref, p1_ref, b2_ref, out_ref):
    acc = p0_ref[...] + p1_ref[...]
    denom = acc[:, 64:65]
    db = jnp.where(denom > 0, denom, 1.0)
    h2 = acc[:, 0:64] / db + b2_ref[...]
    rm = jnp.max(h2, axis=1, keepdims=True)
    s = h2 - rm
    lse = jnp.log(jnp.sum(jnp.exp(s), axis=1, keepdims=True))
    out_ref[...] = s - lse


def _stage3(p0, p1, b2):
    return pl.pallas_call(
        _k3_body,
        grid=(GRID,),
        in_specs=[
            pl.BlockSpec((BLK, WROW), lambda i: (i, 0)),
            pl.BlockSpec((BLK, WROW), lambda i: (i, 0)),
            pl.BlockSpec((1, 64), lambda i: (0, 0)),
        ],
        out_specs=pl.BlockSpec((BLK, 64), lambda i: (i, 0)),
        out_shape=jax.ShapeDtypeStruct((NPAD, 64), _f32),
    )(p0, p1, b2)


def _vgather(vec, idx):
    return lax.gather(
        vec, idx[:, None],
        lax.GatherDimensionNumbers(offset_dims=(), collapsed_slice_dims=(0,),
                                   start_index_map=(0,)),
        slice_sizes=(1,),
        mode=lax.GatherScatterMode.PROMISE_IN_BOUNDS)


@functools.lru_cache(maxsize=None)
def _make_edge_pass(heads):
    mesh = plsc.VectorSubcoreMesh(core_axis_name="c", subcore_axis_name="s",
                                  num_cores=2, num_subcores=16)
    NBUF = 4

    @functools.partial(
        pl.kernel,
        out_type=jax.ShapeDtypeStruct((2 * NPAD, WROW), _f32),
        mesh=mesh,
        scratch_types=[
            pltpu.VMEM_SHARED((NPAD, WROW), _f32),
            pltpu.VMEM((NBATCH, BB), jnp.int32),
            pltpu.VMEM((NBATCH, BB), jnp.int32),
            [pltpu.VMEM((BB, WROW), _f32) for _ in range(NBUF)],
            [pltpu.VMEM((BB, DROW), _f32) for _ in range(NBUF)],
            pltpu.VMEM((16,), _f32),
            [pltpu.SemaphoreType.DMA for _ in range(NBUF)],
            [pltpu.SemaphoreType.DMA for _ in range(NBUF)],
            [pltpu.SemaphoreType.DMA for _ in range(NBUF)],
        ],
        compiler_params=pltpu.CompilerParams(use_tc_tiling_on_sc=False),
    )
    def edge_pass(src_hbm, dst_hbm, ts_hbm, td_hbm, cv_hbm, z_hbm, out_hbm,
                  acc, src_i, dst_i, s_bufs, d_bufs, c_v, sem_s, sem_d, sem_sc):
        cid = lax.axis_index("c")
        sid = lax.axis_index("s")
        wid = sid * 2 + cid
        r0 = sid * RPT
        pltpu.sync_copy(z_hbm.at[pl.ds(r0, RPT)], acc.at[pl.ds(r0, RPT)])
        pltpu.sync_copy(cv_hbm, c_v)
        pltpu.sync_copy(src_hbm.at[pl.ds(wid * NBATCH, NBATCH)], src_i)
        pltpu.sync_copy(dst_hbm.at[pl.ds(wid * NBATCH, NBATCH)], dst_i)
        plsc.subcore_barrier()
        cvec = c_v[...]
        iota = lax.iota(jnp.int32, 16)
        half = jnp.right_shift(iota, 3)
        colh = jnp.bitwise_and(iota, 7)
        zero16 = iota - iota
        hmc = [2 * t + half for t in range(4)]
        hmc.append(jnp.where(iota < 8, iota, 0))

        def g_start(k, b):
            pltpu.async_copy(ts_hbm.at[src_i.at[k]], s_bufs[b], sem_s[b])
            pltpu.async_copy(td_hbm.at[dst_i.at[k]], d_bufs[b], sem_d[b])

        def g_wait(k, b):
            pltpu.make_async_copy(ts_hbm.at[src_i.at[k]], s_bufs[b],
                                  sem_s[b]).wait()
            pltpu.make_async_copy(td_hbm.at[dst_i.at[k]], d_bufs[b],
                                  sem_d[b]).wait()

        def sc_start(k, b):
            pltpu.async_copy(s_bufs[b], acc.at[dst_i.at[k]], sem_sc[b],
                             add=True)

        def sc_wait(k, b):
            pltpu.make_async_copy(s_bufs[b], acc.at[dst_i.at[k]],
                                  sem_sc[b]).wait()

        def compute(b):
            s_v = s_bufs[b]
            d_v = d_bufs[b]
            if heads == 8:
                @plsc.parallel_loop(0, BB // 2, unroll=2)
                def mbody(j):
                    e0 = 2 * j
                    e1 = 2 * j + 1
                    va0 = s_v[e0, pl.ds(64, 16)]
                    va1 = s_v[e1, pl.ds(64, 16)]
                    vd0 = d_v[e0, pl.ds(0, 16)]
                    vd1 = d_v[e1, pl.ds(0, 16)]
                    a_s = jnp.where(iota < 8, _vgather(va0, 8 + colh),
                                    _vgather(va1, 8 + colh))
                    a_d = jnp.where(iota < 8, _vgather(vd0, colh),
                                    _vgather(vd1, colh))
                    e = a_s + a_d
                    e = jnp.maximum(e, 0.2 * e)
                    w = jnp.exp(e - cvec)
                    for bb, off in ((e0, 0), (e1, 8)):
                        for t in range(5):
                            wv = _vgather(w, off + hmc[t])
                            s_v[bb, pl.ds(16 * t, 16)] = (
                                s_v[bb, pl.ds(16 * t, 16)] * wv)
            else:
                @plsc.parallel_loop(0, BB, unroll=4)
                def mbody(bb):
                    va = s_v[bb, pl.ds(64, 16)]
                    vd = d_v[bb, pl.ds(0, 16)]
                    e = _vgather(va, zero16 + 8) + _vgather(vd, zero16)
                    e = jnp.maximum(e, 0.2 * e)
                    w = jnp.exp(e - cvec)
                    for t in range(5):
                        s_v[bb, pl.ds(16 * t, 16)] = (
                            s_v[bb, pl.ds(16 * t, 16)] * w)

        g_start(0, 0)
        g_start(1, 1)
        MLAST = NBATCH // NBUF - 1

        def mloop(m, carry):
            for b in range(NBUF):
                k = NBUF * m + b
                g_wait(k, b)
                nb = (b + 2) % NBUF
                if b < 2:
                    @pl.when(m > 0)
                    def _():
                        sc_wait(k - 2, nb)
                    g_start(k + 2, nb)
                else:
                    sc_wait(k - 2, nb)

                    @pl.when(m < MLAST)
                    def _():
                        g_start(k + 2, nb)
                compute(b)
                sc_start(k, b)
            return carry

        lax.fori_loop(0, NBATCH // NBUF, mloop, 0)
        sc_wait(NBATCH - 2, 2)
        sc_wait(NBATCH - 1, 3)
        plsc.subcore_barrier()
        pltpu.sync_copy(acc.at[pl.ds(r0, RPT)],
                        out_hbm.at[pl.ds(cid * NPAD + r0, RPT)])

    return edge_pass


def _lrelu(v):
    return jnp.maximum(v, 0.2 * v)


def kernel(x, edge_index, W1, a1_src, a1_dst, b1, W2, a2_src, a2_dst, b2):
    xp = jnp.pad(x, ((0, NPAD - NN), (0, 0)))
    loops = jnp.arange(NN, dtype=edge_index.dtype)
    ei = jnp.concatenate([edge_index, jnp.stack([loops, loops])], axis=1)
    src = jnp.pad(ei[0], (0, EPAD - ETOT),
                  constant_values=NN).reshape(32 * NBATCH, BB)
    dst = jnp.pad(ei[1], (0, EPAD - ETOT),
                  constant_values=NN).reshape(32 * NBATCH, BB)
    blockmask = (lax.broadcasted_iota(jnp.int32, (64, 8), 0) // 8 ==
                 lax.broadcasted_iota(jnp.int32, (64, 8), 1))
    a1s = jnp.where(blockmask, a1_src.reshape(64, 1), 0.0)
    a1d = jnp.where(blockmask, a1_dst.reshape(64, 1), 0.0)
    a2s = jnp.pad(a2_src.reshape(64, 1), ((0, 0), (0, 7)))
    a2d = jnp.pad(a2_dst.reshape(64, 1), ((0, 0), (0, 7)))
    zrows = jnp.zeros((NPAD, WROW), _f32)

    ts1, td1, mx1 = _stage1(xp, W1, a1s, a1d)
    cv1 = jnp.tile(_lrelu(mx1[0, 0:8] + mx1[0, 8:16]), 2)
    part1 = _make_edge_pass(8)(src, dst, ts1, td1, cv1, zrows)

    embf, ts2, td2, mx2 = _stage2(part1[:NPAD], part1[NPAD:],
                                  b1.reshape(1, 64), W2, a2s, a2d)
    cv2 = jnp.full((16,), _lrelu(mx2[0, 0] + mx2[0, 8]), _f32)
    part2 = _make_edge_pass(1)(src, dst, ts2, td2, cv2, zrows)

    logp = _stage3(part2[:NPAD], part2[NPAD:], b2.reshape(1, 64))
    return (embf[:NN], logp[:NN])


# speedup vs baseline: 1.0179x; 1.0179x over previous
import functools

import jax
import jax.numpy as jnp
from jax import lax
from jax.experimental import pallas as pl
from jax.experimental.pallas import tpu as pltpu
from jax.experimental.pallas import tpu_sc as plsc

NN = 10000
NPAD = 10240
RPT = NPAD // 16
NE = 320000
ETOT = NE + NN
EPT = 10752
EPAD = EPT * 32
BB = 128
NBATCH = EPT // BB
WROW = 80
DROW = 16
BLK = 1024
GRID = NPAD // BLK

_f32 = jnp.float32


def _k1_body(x_ref, w1_ref, a1s_ref, a1d_ref, ts_ref, td_ref, mx_ref):
    h = jnp.dot(x_ref[...], w1_ref[...], preferred_element_type=_f32)
    asv = jnp.dot(h, a1s_ref[...], preferred_element_type=_f32)
    adv = jnp.dot(h, a1d_ref[...], preferred_element_type=_f32)
    ts_ref[...] = jnp.concatenate([h, jnp.ones((BLK, 8), _f32), asv], axis=1)
    td_ref[...] = jnp.concatenate([adv, jnp.zeros((BLK, 8), _f32)], axis=1)
    m = jnp.max(jnp.concatenate([asv, adv], axis=1), axis=0, keepdims=True)
    mfull = jnp.concatenate(
        [jnp.broadcast_to(m, (8, 16)), jnp.zeros((8, 112), _f32)], axis=1)
    i = pl.program_id(0)

    @pl.when(i == 0)
    def _():
        mx_ref[...] = mfull

    @pl.when(i != 0)
    def _():
        mx_ref[...] = jnp.maximum(mx_ref[...], mfull)


def _stage1(xp, w1, a1s, a1d):
    return pl.pallas_call(
        _k1_body,
        grid=(GRID,),
        in_specs=[
            pl.BlockSpec((BLK, 128), lambda i: (i, 0)),
            pl.BlockSpec((128, 64), lambda i: (0, 0)),
            pl.BlockSpec((64, 8), lambda i: (0, 0)),
            pl.BlockSpec((64, 8), lambda i: (0, 0)),
        ],
        out_specs=[
            pl.BlockSpec((BLK, WROW), lambda i: (i, 0)),
            pl.BlockSpec((BLK, DROW), lambda i: (i, 0)),
            pl.BlockSpec((8, 128), lambda i: (0, 0)),
        ],
        out_shape=[
            jax.ShapeDtypeStruct((NPAD, WROW), _f32),
            jax.ShapeDtypeStruct((NPAD, DROW), _f32),
            jax.ShapeDtypeStruct((8, 128), _f32),
        ],
    )(xp, w1, a1s, a1d)


def _k2_body(p0_ref, p1_ref, b1_ref, w2_ref, a2s_ref, a2d_ref,
             emb_ref, ts_ref, td_ref, mx_ref):
    acc = p0_ref[...] + p1_ref[...]
    numer = acc[:, 0:64]
    denom = acc[:, 64:72]
    r8 = (lax.broadcasted_iota(jnp.int32, (8, 64), 1) // 8 ==
          lax.broadcasted_iota(jnp.int32, (8, 64), 0)).astype(_f32)
    db = jnp.dot(denom, r8, preferred_element_type=_f32)
    db = jnp.where(db > 0, db, 1.0)
    out1 = numer / db + b1_ref[...]
    emb_ref[...] = out1
    hact = jnp.where(out1 > 0, out1, jnp.exp(jnp.minimum(out1, 0.0)) - 1.0)
    g = jnp.dot(hact, w2_ref[...], preferred_element_type=_f32)
    asv = jnp.dot(g, a2s_ref[...], preferred_element_type=_f32)
    adv = jnp.dot(g, a2d_ref[...], preferred_element_type=_f32)
    ts_ref[...] = jnp.concatenate([g, jnp.ones((BLK, 8), _f32), asv], axis=1)
    td_ref[...] = jnp.concatenate([adv, jnp.zeros((BLK, 8), _f32)], axis=1)
    m = jnp.max(jnp.concatenate([asv, adv], axis=1), axis=0, keepdims=True)
    mfull = jnp.concatenate(
        [jnp.broadcast_to(m, (8, 16)), jnp.zeros((8, 112), _f32)], axis=1)
    i = pl.program_id(0)

    @pl.when(i == 0)
    def _():
        mx_ref[...] = mfull

    @pl.when(i != 0)
    def _():
        mx_ref[...] = jnp.maximum(mx_ref[...], mfull)


def _stage2(p0, p1, b1, w2, a2s, a2d):
    return pl.pallas_call(
        _k2_body,
        grid=(GRID,),
        in_specs=[
            pl.BlockSpec((BLK, WROW), lambda i: (i, 0)),
            pl.BlockSpec((BLK, WROW), lambda i: (i, 0)),
            pl.BlockSpec((1, 64), lambda i: (0, 0)),
            pl.BlockSpec((64, 64), lambda i: (0, 0)),
            pl.BlockSpec((64, 8), lambda i: (0, 0)),
            pl.BlockSpec((64, 8), lambda i: (0, 0)),
        ],
        out_specs=[
            pl.BlockSpec((BLK, 64), lambda i: (i, 0)),
            pl.BlockSpec((BLK, WROW), lambda i: (i, 0)),
            pl.BlockSpec((BLK, DROW), lambda i: (i, 0)),
            pl.BlockSpec((8, 128), lambda i: (0, 0)),
        ],
        out_shape=[
            jax.ShapeDtypeStruct((NPAD, 64), _f32),
            jax.ShapeDtypeStruct((NPAD, WROW), _f32),
            jax.ShapeDtypeStruct((NPAD, DROW), _f32),
            jax.ShapeDtypeStruct((8, 128), _f32),
        ],
    )(p0, p1, b1, w2, a2s, a2d)


def _k3_body(p0_ref, p1_ref, b2_ref, out_ref):
    acc = p0_ref[...] + p1_ref[...]
    denom = acc[:, 64:65]
    db = jnp.where(denom > 0, denom, 1.0)
    h2 = acc[:, 0:64] / db + b2_ref[...]
    rm = jnp.max(h2, axis=1, keepdims=True)
    s = h2 - rm
    lse = jnp.log(jnp.sum(jnp.exp(s), axis=1, keepdims=True))
    out_ref[...] = s - lse


def _stage3(p0, p1, b2):
    return pl.pallas_call(
        _k3_body,
        grid=(GRID,),
        in_specs=[
            pl.BlockSpec((BLK, WROW), lambda i: (i, 0)),
            pl.BlockSpec((BLK, WROW), lambda i: (i, 0)),
            pl.BlockSpec((1, 64), lambda i: (0, 0)),
        ],
        out_specs=pl.BlockSpec((BLK, 64), lambda i: (i, 0)),
        out_shape=jax.ShapeDtypeStruct((NPAD, 64), _f32),
    )(p0, p1, b2)


def _vgather(vec, idx):
    return lax.gather(
        vec, idx[:, None],
        lax.GatherDimensionNumbers(offset_dims=(), collapsed_slice_dims=(0,),
                                   start_index_map=(0,)),
        slice_sizes=(1,),
        mode=lax.GatherScatterMode.PROMISE_IN_BOUNDS)


@functools.lru_cache(maxsize=None)
def _make_edge_pass(heads):
    mesh = plsc.VectorSubcoreMesh(core_axis_name="c", subcore_axis_name="s",
                                  num_cores=2, num_subcores=16)
    NBUF = 4

    @functools.partial(
        pl.kernel,
        out_type=jax.ShapeDtypeStruct((2 * NPAD, WROW), _f32),
        mesh=mesh,
        scratch_types=[
            pltpu.VMEM_SHARED((NPAD, WROW), _f32),
            pltpu.VMEM((NBATCH, BB), jnp.int32),
            pltpu.VMEM((NBATCH, BB), jnp.int32),
            [pltpu.VMEM((BB, WROW), _f32) for _ in range(NBUF)],
            [pltpu.VMEM((BB, DROW), _f32) for _ in range(NBUF)],
            pltpu.VMEM((16,), _f32),
            [pltpu.SemaphoreType.DMA for _ in range(NBUF)],
            [pltpu.SemaphoreType.DMA for _ in range(NBUF)],
            [pltpu.SemaphoreType.DMA for _ in range(NBUF)],
        ],
        compiler_params=pltpu.CompilerParams(use_tc_tiling_on_sc=False),
    )
    def edge_pass(src_hbm, dst_hbm, ts_hbm, td_hbm, cv_hbm, z_hbm, out_hbm,
                  acc, src_i, dst_i, s_bufs, d_bufs, c_v, sem_s, sem_d, sem_sc):
        cid = lax.axis_index("c")
        sid = lax.axis_index("s")
        wid = sid * 2 + cid
        r0 = sid * RPT
        pltpu.sync_copy(z_hbm.at[pl.ds(r0, RPT)], acc.at[pl.ds(r0, RPT)])
        pltpu.sync_copy(cv_hbm, c_v)
        pltpu.sync_copy(src_hbm.at[pl.ds(wid * NBATCH, NBATCH)], src_i)
        pltpu.sync_copy(dst_hbm.at[pl.ds(wid * NBATCH, NBATCH)], dst_i)
        plsc.subcore_barrier()
        cvec = c_v[...]
        iota = lax.iota(jnp.int32, 16)
        half = jnp.right_shift(iota, 3)
        colh = jnp.bitwise_and(iota, 7)
        zero16 = iota - iota
        hmc = [2 * t + half for t in range(4)]
        hmc.append(jnp.where(iota < 8, iota, 0))

        def g_start(k, b):
            pltpu.async_copy(ts_hbm.at[src_i.at[k]], s_bufs[b], sem_s[b])
            pltpu.async_copy(td_hbm.at[dst_i.at[k]], d_bufs[b], sem_d[b])

        def g_wait(k, b):
            pltpu.make_async_copy(ts_hbm.at[src_i.at[k]], s_bufs[b],
                                  sem_s[b]).wait()
            pltpu.make_async_copy(td_hbm.at[dst_i.at[k]], d_bufs[b],
                                  sem_d[b]).wait()

        def sc_start(k, b):
            pltpu.async_copy(s_bufs[b], acc.at[dst_i.at[k]], sem_sc[b],
                             add=True)

        def sc_wait(k, b):
            pltpu.make_async_copy(s_bufs[b], acc.at[dst_i.at[k]],
                                  sem_sc[b]).wait()

        def compute(b):
            s_v = s_bufs[b]
            d_v = d_bufs[b]
            if heads == 8:
                def mbody(j, c2):
                    e0 = 2 * j
                    e1 = 2 * j + 1
                    va0 = s_v[e0, pl.ds(64, 16)]
                    va1 = s_v[e1, pl.ds(64, 16)]
                    vd0 = d_v[e0, pl.ds(0, 16)]
                    vd1 = d_v[e1, pl.ds(0, 16)]
                    a_s = jnp.where(iota < 8, _vgather(va0, 8 + colh),
                                    _vgather(va1, 8 + colh))
                    a_d = jnp.where(iota < 8, _vgather(vd0, colh),
                                    _vgather(vd1, colh))
                    e = a_s + a_d
                    e = jnp.maximum(e, 0.2 * e)
                    w = jnp.exp(e - cvec)
                    for bb, off in ((e0, 0), (e1, 8)):
                        for t in range(5):
                            wv = _vgather(w, off + hmc[t])
                            s_v[bb, pl.ds(16 * t, 16)] = (
                                s_v[bb, pl.ds(16 * t, 16)] * wv)
                    return c2
                lax.fori_loop(0, BB // 2, mbody, 0)
            else:
                def mbody(bb, c2):
                    va = s_v[bb, pl.ds(64, 16)]
                    vd = d_v[bb, pl.ds(0, 16)]
                    e = _vgather(va, zero16 + 8) + _vgather(vd, zero16)
                    e = jnp.maximum(e, 0.2 * e)
                    w = jnp.exp(e - cvec)
                    for t in range(5):
                        s_v[bb, pl.ds(16 * t, 16)] = (
                            s_v[bb, pl.ds(16 * t, 16)] * w)
                    return c2
                lax.fori_loop(0, BB, mbody, 0)

        g_start(0, 0)
        g_start(1, 1)
        MLAST = NBATCH // NBUF - 1

        def mloop(m, carry):
            for b in range(NBUF):
                k = NBUF * m + b
                g_wait(k, b)
                nb = (b + 2) % NBUF
                if b < 2:
                    @pl.when(m > 0)
                    def _():
                        sc_wait(k - 2, nb)
                    g_start(k + 2, nb)
                else:
                    sc_wait(k - 2, nb)

                    @pl.when(m < MLAST)
                    def _():
                        g_start(k + 2, nb)
                compute(b)
                sc_start(k, b)
            return carry

        lax.fori_loop(0, NBATCH // NBUF, mloop, 0)
        sc_wait(NBATCH - 2, 2)
        sc_wait(NBATCH - 1, 3)
        plsc.subcore_barrier()
        pltpu.sync_copy(acc.at[pl.ds(r0, RPT)],
                        out_hbm.at[pl.ds(cid * NPAD + r0, RPT)])

    return edge_pass


def _lrelu(v):
    return jnp.maximum(v, 0.2 * v)


def kernel(x, edge_index, W1, a1_src, a1_dst, b1, W2, a2_src, a2_dst, b2):
    xp = jnp.pad(x, ((0, NPAD - NN), (0, 0)))
    loops = jnp.arange(NN, dtype=edge_index.dtype)
    ei = jnp.concatenate([edge_index, jnp.stack([loops, loops])], axis=1)
    src = jnp.pad(ei[0], (0, EPAD - ETOT),
                  constant_values=NN).reshape(32 * NBATCH, BB)
    dst = jnp.pad(ei[1], (0, EPAD - ETOT),
                  constant_values=NN).reshape(32 * NBATCH, BB)
    blockmask = (lax.broadcasted_iota(jnp.int32, (64, 8), 0) // 8 ==
                 lax.broadcasted_iota(jnp.int32, (64, 8), 1))
    a1s = jnp.where(blockmask, a1_src.reshape(64, 1), 0.0)
    a1d = jnp.where(blockmask, a1_dst.reshape(64, 1), 0.0)
    a2s = jnp.pad(a2_src.reshape(64, 1), ((0, 0), (0, 7)))
    a2d = jnp.pad(a2_dst.reshape(64, 1), ((0, 0), (0, 7)))
    zrows = jnp.zeros((NPAD, WROW), _f32)

    ts1, td1, mx1 = _stage1(xp, W1, a1s, a1d)
    cv1 = jnp.tile(_lrelu(mx1[0, 0:8] + mx1[0, 8:16]), 2)
    part1 = _make_edge_pass(8)(src, dst, ts1, td1, cv1, zrows)

    embf, ts2, td2, mx2 = _stage2(part1[:NPAD], part1[NPAD:],
                                  b1.reshape(1, 64), W2, a2s, a2d)
    cv2 = jnp.full((16,), _lrelu(mx2[0, 0] + mx2[0, 8]), _f32)
    part2 = _make_edge_pass(1)(src, dst, ts2, td2, cv2, zrows)

    logp = _stage3(part2[:NPAD], part2[NPAD:], b2.reshape(1, 64))
    return (embf[:NN], logp[:NN])


# speedup vs baseline: 1.7227x; 1.6924x over previous
import functools

import jax
import jax.numpy as jnp
from jax import lax
from jax.experimental import pallas as pl
from jax.experimental.pallas import tpu as pltpu
from jax.experimental.pallas import tpu_sc as plsc

NN = 10000
NPAD = 10240
RPT = NPAD // 16
NE = 320000
ETOT = NE + NN
EPT = 10368
EPAD = EPT * 32
BB = 96
NBATCH = EPT // BB
WROW = 80
DROW = 16
BLK = 1024
GRID = NPAD // BLK

_f32 = jnp.float32


def _k1_body(x_ref, w1_ref, a1s_ref, a1d_ref, ts_ref, td_ref, mx_ref):
    h = jnp.dot(x_ref[...], w1_ref[...], preferred_element_type=_f32)
    asv = jnp.dot(h, a1s_ref[...], preferred_element_type=_f32)
    adv = jnp.dot(h, a1d_ref[...], preferred_element_type=_f32)
    ts_ref[...] = jnp.concatenate([h, jnp.ones((BLK, 8), _f32), asv], axis=1)
    td_ref[...] = jnp.concatenate([adv, jnp.zeros((BLK, 8), _f32)], axis=1)
    m = jnp.max(jnp.concatenate([asv, adv], axis=1), axis=0, keepdims=True)
    mfull = jnp.concatenate(
        [jnp.broadcast_to(m, (8, 16)), jnp.zeros((8, 112), _f32)], axis=1)
    i = pl.program_id(0)

    @pl.when(i == 0)
    def _():
        mx_ref[...] = mfull

    @pl.when(i != 0)
    def _():
        mx_ref[...] = jnp.maximum(mx_ref[...], mfull)


def _stage1(xp, w1, a1s, a1d):
    return pl.pallas_call(
        _k1_body,
        grid=(GRID,),
        in_specs=[
            pl.BlockSpec((BLK, 128), lambda i: (i, 0)),
            pl.BlockSpec((128, 64), lambda i: (0, 0)),
            pl.BlockSpec((64, 8), lambda i: (0, 0)),
            pl.BlockSpec((64, 8), lambda i: (0, 0)),
        ],
        out_specs=[
            pl.BlockSpec((BLK, WROW), lambda i: (i, 0)),
            pl.BlockSpec((BLK, DROW), lambda i: (i, 0)),
            pl.BlockSpec((8, 128), lambda i: (0, 0)),
        ],
        out_shape=[
            jax.ShapeDtypeStruct((NPAD, WROW), _f32),
            jax.ShapeDtypeStruct((NPAD, DROW), _f32),
            jax.ShapeDtypeStruct((8, 128), _f32),
        ],
    )(xp, w1, a1s, a1d)


def _k2_body(p0_ref, p1_ref, b1_ref, w2_ref, a2s_ref, a2d_ref,
             emb_ref, ts_ref, td_ref, mx_ref):
    acc = p0_ref[...] + p1_ref[...]
    numer = acc[:, 0:64]
    denom = acc[:, 64:72]
    r8 = (lax.broadcasted_iota(jnp.int32, (8, 64), 1) // 8 ==
          lax.broadcasted_iota(jnp.int32, (8, 64), 0)).astype(_f32)
    db = jnp.dot(denom, r8, preferred_element_type=_f32)
    db = jnp.where(db > 0, db, 1.0)
    out1 = numer / db + b1_ref[...]
    emb_ref[...] = out1
    hact = jnp.where(out1 > 0, out1, jnp.exp(jnp.minimum(out1, 0.0)) - 1.0)
    g = jnp.dot(hact, w2_ref[...], preferred_element_type=_f32)
    asv = jnp.dot(g, a2s_ref[...], preferred_element_type=_f32)
    adv = jnp.dot(g, a2d_ref[...], preferred_element_type=_f32)
    ts_ref[...] = jnp.concatenate([g, jnp.ones((BLK, 8), _f32), asv], axis=1)
    td_ref[...] = jnp.concatenate([adv, jnp.zeros((BLK, 8), _f32)], axis=1)
    m = jnp.max(jnp.concatenate([asv, adv], axis=1), axis=0, keepdims=True)
    mfull = jnp.concatenate(
        [jnp.broadcast_to(m, (8, 16)), jnp.zeros((8, 112), _f32)], axis=1)
    i = pl.program_id(0)

    @pl.when(i == 0)
    def _():
        mx_ref[...] = mfull

    @pl.when(i != 0)
    def _():
        mx_ref[...] = jnp.maximum(mx_ref[...], mfull)


def _stage2(p0, p1, b1, w2, a2s, a2d):
    return pl.pallas_call(
        _k2_body,
        grid=(GRID,),
        in_specs=[
            pl.BlockSpec((BLK, WROW), lambda i: (i, 0)),
            pl.BlockSpec((BLK, WROW), lambda i: (i, 0)),
            pl.BlockSpec((1, 64), lambda i: (0, 0)),
            pl.BlockSpec((64, 64), lambda i: (0, 0)),
            pl.BlockSpec((64, 8), lambda i: (0, 0)),
            pl.BlockSpec((64, 8), lambda i: (0, 0)),
        ],
        out_specs=[
            pl.BlockSpec((BLK, 64), lambda i: (i, 0)),
            pl.BlockSpec((BLK, WROW), lambda i: (i, 0)),
            pl.BlockSpec((BLK, DROW), lambda i: (i, 0)),
            pl.BlockSpec((8, 128), lambda i: (0, 0)),
        ],
        out_shape=[
            jax.ShapeDtypeStruct((NPAD, 64), _f32),
            jax.ShapeDtypeStruct((NPAD, WROW), _f32),
            jax.ShapeDtypeStruct((NPAD, DROW), _f32),
            jax.ShapeDtypeStruct((8, 128), _f32),
        ],
    )(p0, p1, b1, w2, a2s, a2d)


def _k3_body(p0_ref, p1_ref, b2_ref, out_ref):
    acc = p0_ref[...] + p1_ref[...]
    denom = acc[:, 64:65]
    db = jnp.where(denom > 0, denom, 1.0)
    h2 = acc[:, 0:64] / db + b2_ref[...]
    rm = jnp.max(h2, axis=1, keepdims=True)
    s = h2 - rm
    lse = jnp.log(jnp.sum(jnp.exp(s), axis=1, keepdims=True))
    out_ref[...] = s - lse


def _stage3(p0, p1, b2):
    return pl.pallas_call(
        _k3_body,
        grid=(GRID,),
        in_specs=[
            pl.BlockSpec((BLK, WROW), lambda i: (i, 0)),
            pl.BlockSpec((BLK, WROW), lambda i: (i, 0)),
            pl.BlockSpec((1, 64), lambda i: (0, 0)),
        ],
        out_specs=pl.BlockSpec((BLK, 64), lambda i: (i, 0)),
        out_shape=jax.ShapeDtypeStruct((NPAD, 64), _f32),
    )(p0, p1, b2)


def _vgather(vec, idx):
    return lax.gather(
        vec, idx[:, None],
        lax.GatherDimensionNumbers(offset_dims=(), collapsed_slice_dims=(0,),
                                   start_index_map=(0,)),
        slice_sizes=(1,),
        mode=lax.GatherScatterMode.PROMISE_IN_BOUNDS)


@functools.lru_cache(maxsize=None)
def _make_edge_pass(heads):
    mesh = plsc.VectorSubcoreMesh(core_axis_name="c", subcore_axis_name="s",
                                  num_cores=2, num_subcores=16)
    NBUF = 4

    @functools.partial(
        pl.kernel,
        out_type=jax.ShapeDtypeStruct((2 * NPAD, WROW), _f32),
        mesh=mesh,
        scratch_types=[
            pltpu.VMEM_SHARED((NPAD, WROW), _f32),
            pltpu.VMEM((NBATCH, BB), jnp.int32),
            pltpu.VMEM((NBATCH, BB), jnp.int32),
            [pltpu.VMEM((BB, WROW), _f32) for _ in range(NBUF)],
            [pltpu.VMEM((BB, DROW), _f32) for _ in range(NBUF)],
            pltpu.VMEM((16,), _f32),
            [pltpu.SemaphoreType.DMA for _ in range(NBUF)],
            [pltpu.SemaphoreType.DMA for _ in range(NBUF)],
            [pltpu.SemaphoreType.DMA for _ in range(NBUF)],
        ],
        compiler_params=pltpu.CompilerParams(use_tc_tiling_on_sc=False),
    )
    def edge_pass(src_hbm, dst_hbm, ts_hbm, td_hbm, cv_hbm, z_hbm, out_hbm,
                  acc, src_i, dst_i, s_bufs, d_bufs, c_v, sem_s, sem_d, sem_sc):
        cid = lax.axis_index("c")
        sid = lax.axis_index("s")
        wid = sid * 2 + cid
        r0 = sid * RPT
        pltpu.sync_copy(z_hbm.at[pl.ds(r0, RPT)], acc.at[pl.ds(r0, RPT)])
        pltpu.sync_copy(cv_hbm, c_v)
        pltpu.sync_copy(src_hbm.at[pl.ds(wid * NBATCH, NBATCH)], src_i)
        pltpu.sync_copy(dst_hbm.at[pl.ds(wid * NBATCH, NBATCH)], dst_i)
        plsc.subcore_barrier()
        cvec = c_v[...]
        iota = lax.iota(jnp.int32, 16)
        half = jnp.right_shift(iota, 3)
        colh = jnp.bitwise_and(iota, 7)
        zero16 = iota - iota
        hmc = [2 * t + half for t in range(4)]
        hmc.append(jnp.where(iota < 8, iota, 0))

        def g_start(k, b):
            pltpu.async_copy(ts_hbm.at[src_i.at[k]], s_bufs[b], sem_s[b])
            pltpu.async_copy(td_hbm.at[dst_i.at[k]], d_bufs[b], sem_d[b])

        def g_wait(k, b):
            pltpu.make_async_copy(ts_hbm.at[src_i.at[k]], s_bufs[b],
                                  sem_s[b]).wait()
            pltpu.make_async_copy(td_hbm.at[dst_i.at[k]], d_bufs[b],
                                  sem_d[b]).wait()

        def sc_start(k, b):
            pltpu.async_copy(s_bufs[b], acc.at[dst_i.at[k]], sem_sc[b],
                             add=True)

        def sc_wait(k, b):
            pltpu.make_async_copy(s_bufs[b], acc.at[dst_i.at[k]],
                                  sem_sc[b]).wait()

        def compute(b):
            s_v = s_bufs[b]
            d_v = d_bufs[b]
            if heads == 8:
                def mbody(j, c2):
                    e0 = 2 * j
                    e1 = 2 * j + 1
                    va0 = s_v[e0, pl.ds(64, 16)]
                    va1 = s_v[e1, pl.ds(64, 16)]
                    vd0 = d_v[e0, pl.ds(0, 16)]
                    vd1 = d_v[e1, pl.ds(0, 16)]
                    a_s = jnp.where(iota < 8, _vgather(va0, 8 + colh),
                                    _vgather(va1, 8 + colh))
                    a_d = jnp.where(iota < 8, _vgather(vd0, colh),
                                    _vgather(vd1, colh))
                    e = a_s + a_d
                    e = jnp.maximum(e, 0.2 * e)
                    w = jnp.exp(e - cvec)
                    for bb, off in ((e0, 0), (e1, 8)):
                        for t in range(5):
                            wv = _vgather(w, off + hmc[t])
                            s_v[bb, pl.ds(16 * t, 16)] = (
                                s_v[bb, pl.ds(16 * t, 16)] * wv)
                    return c2
                lax.fori_loop(0, BB // 2, mbody, 0)
            else:
                def mbody(bb, c2):
                    va = s_v[bb, pl.ds(64, 16)]
                    vd = d_v[bb, pl.ds(0, 16)]
                    e = _vgather(va, zero16 + 8) + _vgather(vd, zero16)
                    e = jnp.maximum(e, 0.2 * e)
                    w = jnp.exp(e - cvec)
                    for t in range(5):
                        s_v[bb, pl.ds(16 * t, 16)] = (
                            s_v[bb, pl.ds(16 * t, 16)] * w)
                    return c2
                lax.fori_loop(0, BB, mbody, 0)

        g_start(0, 0)
        g_start(1, 1)
        MLAST = NBATCH // NBUF - 1

        def mloop(m, carry):
            for b in range(NBUF):
                k = NBUF * m + b
                g_wait(k, b)
                nb = (b + 2) % NBUF
                if b < 2:
                    @pl.when(m > 0)
                    def _():
                        sc_wait(k - 2, nb)
                    g_start(k + 2, nb)
                else:
                    sc_wait(k - 2, nb)

                    @pl.when(m < MLAST)
                    def _():
                        g_start(k + 2, nb)
                compute(b)
                sc_start(k, b)
            return carry

        lax.fori_loop(0, NBATCH // NBUF, mloop, 0)
        sc_wait(NBATCH - 2, 2)
        sc_wait(NBATCH - 1, 3)
        plsc.subcore_barrier()
        pltpu.sync_copy(acc.at[pl.ds(r0, RPT)],
                        out_hbm.at[pl.ds(cid * NPAD + r0, RPT)])

    return edge_pass


def _lrelu(v):
    return jnp.maximum(v, 0.2 * v)


def kernel(x, edge_index, W1, a1_src, a1_dst, b1, W2, a2_src, a2_dst, b2):
    xp = jnp.pad(x, ((0, NPAD - NN), (0, 0)))
    loops = jnp.arange(NN, dtype=edge_index.dtype)
    ei = jnp.concatenate([edge_index, jnp.stack([loops, loops])], axis=1)
    src = jnp.pad(ei[0], (0, EPAD - ETOT),
                  constant_values=NN).reshape(32 * NBATCH, BB)
    dst = jnp.pad(ei[1], (0, EPAD - ETOT),
                  constant_values=NN).reshape(32 * NBATCH, BB)
    blockmask = (lax.broadcasted_iota(jnp.int32, (64, 8), 0) // 8 ==
                 lax.broadcasted_iota(jnp.int32, (64, 8), 1))
    a1s = jnp.where(blockmask, a1_src.reshape(64, 1), 0.0)
    a1d = jnp.where(blockmask, a1_dst.reshape(64, 1), 0.0)
    a2s = jnp.pad(a2_src.reshape(64, 1), ((0, 0), (0, 7)))
    a2d = jnp.pad(a2_dst.reshape(64, 1), ((0, 0), (0, 7)))
    zrows = jnp.zeros((NPAD, WROW), _f32)

    ts1, td1, mx1 = _stage1(xp, W1, a1s, a1d)
    cv1 = jnp.tile(_lrelu(mx1[0, 0:8] + mx1[0, 8:16]), 2)
    part1 = _make_edge_pass(8)(src, dst, ts1, td1, cv1, zrows)

    embf, ts2, td2, mx2 = _stage2(part1[:NPAD], part1[NPAD:],
                                  b1.reshape(1, 64), W2, a2s, a2d)
    cv2 = jnp.full((16,), _lrelu(mx2[0, 0] + mx2[0, 8]), _f32)
    part2 = _make_edge_pass(1)(src, dst, ts2, td2, cv2, zrows)

    logp = _stage3(part2[:NPAD], part2[NPAD:], b2.reshape(1, 64))
    return (embf[:NN], logp[:NN])


# speedup vs baseline: 2.2777x; 1.3222x over previous
import functools

import jax
import jax.numpy as jnp
from jax import lax
from jax.experimental import pallas as pl
from jax.experimental.pallas import tpu as pltpu
from jax.experimental.pallas import tpu_sc as plsc

NN = 10000
NPAD = 10240
RPT = NPAD // 16
NE = 320000
ETOT = NE + NN
EPT = 10368
EPAD = EPT * 32
BB = 96
NBATCH = EPT // BB
WROW = 80
DROW = 16
BLK = 1024
GRID = NPAD // BLK

_f32 = jnp.float32


def _k1_body(x_ref, w1_ref, a1s_ref, a1d_ref, ts_ref, td_ref, mx_ref):
    h = jnp.dot(x_ref[...], w1_ref[...], preferred_element_type=_f32)
    asv = jnp.dot(h, a1s_ref[...], preferred_element_type=_f32)
    adv = jnp.dot(h, a1d_ref[...], preferred_element_type=_f32)
    ts_ref[...] = jnp.concatenate([h, jnp.ones((BLK, 8), _f32), asv], axis=1)
    td_ref[...] = jnp.concatenate([adv, jnp.zeros((BLK, 8), _f32)], axis=1)
    m = jnp.max(jnp.concatenate([asv, adv], axis=1), axis=0, keepdims=True)
    mfull = jnp.concatenate(
        [jnp.broadcast_to(m, (8, 16)), jnp.zeros((8, 112), _f32)], axis=1)
    i = pl.program_id(0)

    @pl.when(i == 0)
    def _():
        mx_ref[...] = mfull

    @pl.when(i != 0)
    def _():
        mx_ref[...] = jnp.maximum(mx_ref[...], mfull)


def _stage1(xp, w1, a1s, a1d):
    return pl.pallas_call(
        _k1_body,
        grid=(GRID,),
        in_specs=[
            pl.BlockSpec((BLK, 128), lambda i: (i, 0)),
            pl.BlockSpec((128, 64), lambda i: (0, 0)),
            pl.BlockSpec((64, 8), lambda i: (0, 0)),
            pl.BlockSpec((64, 8), lambda i: (0, 0)),
        ],
        out_specs=[
            pl.BlockSpec((BLK, WROW), lambda i: (i, 0)),
            pl.BlockSpec((BLK, DROW), lambda i: (i, 0)),
            pl.BlockSpec((8, 128), lambda i: (0, 0)),
        ],
        out_shape=[
            jax.ShapeDtypeStruct((NPAD, WROW), _f32),
            jax.ShapeDtypeStruct((NPAD, DROW), _f32),
            jax.ShapeDtypeStruct((8, 128), _f32),
        ],
    )(xp, w1, a1s, a1d)


def _k2_body(p0_ref, p1_ref, b1_ref, w2_ref, a2s_ref, a2d_ref,
             emb_ref, ts_ref, td_ref, mx_ref):
    acc = p0_ref[...] + p1_ref[...]
    numer = acc[:, 0:64]
    denom = acc[:, 64:72]
    r8 = (lax.broadcasted_iota(jnp.int32, (8, 64), 1) // 8 ==
          lax.broadcasted_iota(jnp.int32, (8, 64), 0)).astype(_f32)
    db = jnp.dot(denom, r8, preferred_element_type=_f32)
    db = jnp.where(db > 0, db, 1.0)
    out1 = numer / db + b1_ref[...]
    emb_ref[...] = out1
    hact = jnp.where(out1 > 0, out1, jnp.exp(jnp.minimum(out1, 0.0)) - 1.0)
    g = jnp.dot(hact, w2_ref[...], preferred_element_type=_f32)
    asv = jnp.dot(g, a2s_ref[...], preferred_element_type=_f32)
    adv = jnp.dot(g, a2d_ref[...], preferred_element_type=_f32)
    ts_ref[...] = jnp.concatenate([g, jnp.ones((BLK, 8), _f32), asv], axis=1)
    td_ref[...] = jnp.concatenate([adv, jnp.zeros((BLK, 8), _f32)], axis=1)
    m = jnp.max(jnp.concatenate([asv, adv], axis=1), axis=0, keepdims=True)
    mfull = jnp.concatenate(
        [jnp.broadcast_to(m, (8, 16)), jnp.zeros((8, 112), _f32)], axis=1)
    i = pl.program_id(0)

    @pl.when(i == 0)
    def _():
        mx_ref[...] = mfull

    @pl.when(i != 0)
    def _():
        mx_ref[...] = jnp.maximum(mx_ref[...], mfull)


def _stage2(p0, p1, b1, w2, a2s, a2d):
    return pl.pallas_call(
        _k2_body,
        grid=(GRID,),
        in_specs=[
            pl.BlockSpec((BLK, WROW), lambda i: (i, 0)),
            pl.BlockSpec((BLK, WROW), lambda i: (i, 0)),
            pl.BlockSpec((1, 64), lambda i: (0, 0)),
            pl.BlockSpec((64, 64), lambda i: (0, 0)),
            pl.BlockSpec((64, 8), lambda i: (0, 0)),
            pl.BlockSpec((64, 8), lambda i: (0, 0)),
        ],
        out_specs=[
            pl.BlockSpec((BLK, 64), lambda i: (i, 0)),
            pl.BlockSpec((BLK, WROW), lambda i: (i, 0)),
            pl.BlockSpec((BLK, DROW), lambda i: (i, 0)),
            pl.BlockSpec((8, 128), lambda i: (0, 0)),
        ],
        out_shape=[
            jax.ShapeDtypeStruct((NPAD, 64), _f32),
            jax.ShapeDtypeStruct((NPAD, WROW), _f32),
            jax.ShapeDtypeStruct((NPAD, DROW), _f32),
            jax.ShapeDtypeStruct((8, 128), _f32),
        ],
    )(p0, p1, b1, w2, a2s, a2d)


def _k3_body(p0_ref, p1_ref, b2_ref, out_ref):
    acc = p0_ref[...] + p1_ref[...]
    denom = acc[:, 64:65]
    db = jnp.where(denom > 0, denom, 1.0)
    h2 = acc[:, 0:64] / db + b2_ref[...]
    rm = jnp.max(h2, axis=1, keepdims=True)
    s = h2 - rm
    lse = jnp.log(jnp.sum(jnp.exp(s), axis=1, keepdims=True))
    out_ref[...] = s - lse


def _stage3(p0, p1, b2):
    return pl.pallas_call(
        _k3_body,
        grid=(GRID,),
        in_specs=[
            pl.BlockSpec((BLK, WROW), lambda i: (i, 0)),
            pl.BlockSpec((BLK, WROW), lambda i: (i, 0)),
            pl.BlockSpec((1, 64), lambda i: (0, 0)),
        ],
        out_specs=pl.BlockSpec((BLK, 64), lambda i: (i, 0)),
        out_shape=jax.ShapeDtypeStruct((NPAD, 64), _f32),
    )(p0, p1, b2)


def _vgather(vec, idx):
    return lax.gather(
        vec, idx[:, None],
        lax.GatherDimensionNumbers(offset_dims=(), collapsed_slice_dims=(0,),
                                   start_index_map=(0,)),
        slice_sizes=(1,),
        mode=lax.GatherScatterMode.PROMISE_IN_BOUNDS)


@functools.lru_cache(maxsize=None)
def _make_edge_pass(heads):
    mesh = plsc.VectorSubcoreMesh(core_axis_name="c", subcore_axis_name="s",
                                  num_cores=2, num_subcores=16)
    NBUF = 4

    @functools.partial(
        pl.kernel,
        out_type=jax.ShapeDtypeStruct((2 * NPAD, WROW), _f32),
        mesh=mesh,
        scratch_types=[
            pltpu.VMEM_SHARED((NPAD, WROW), _f32),
            pltpu.VMEM((NBATCH, BB), jnp.int32),
            pltpu.VMEM((NBATCH, BB), jnp.int32),
            [pltpu.VMEM((BB, WROW), _f32) for _ in range(NBUF)],
            [pltpu.VMEM((BB, DROW), _f32) for _ in range(NBUF)],
            pltpu.VMEM((16,), _f32),
            [pltpu.SemaphoreType.DMA for _ in range(NBUF)],
            [pltpu.SemaphoreType.DMA for _ in range(NBUF)],
            [pltpu.SemaphoreType.DMA for _ in range(NBUF)],
        ],
        compiler_params=pltpu.CompilerParams(use_tc_tiling_on_sc=False),
    )
    def edge_pass(src_hbm, dst_hbm, ts_hbm, td_hbm, cv_hbm, z_hbm, out_hbm,
                  acc, src_i, dst_i, s_bufs, d_bufs, c_v, sem_s, sem_d, sem_sc):
        cid = lax.axis_index("c")
        sid = lax.axis_index("s")
        wid = sid * 2 + cid
        r0 = sid * RPT
        pltpu.sync_copy(z_hbm.at[pl.ds(r0, RPT)], acc.at[pl.ds(r0, RPT)])
        pltpu.sync_copy(cv_hbm, c_v)
        pltpu.sync_copy(src_hbm.at[pl.ds(wid * NBATCH, NBATCH)], src_i)
        pltpu.sync_copy(dst_hbm.at[pl.ds(wid * NBATCH, NBATCH)], dst_i)
        plsc.subcore_barrier()
        cvec = c_v[...]
        iota = lax.iota(jnp.int32, 16)
        half = jnp.right_shift(iota, 3)
        colh = jnp.bitwise_and(iota, 7)
        zero16 = iota - iota
        hmc = [2 * t + half for t in range(4)]
        hmc.append(jnp.where(iota < 8, iota, 0))

        def g_start(k, b):
            pltpu.async_copy(ts_hbm.at[src_i.at[k]], s_bufs[b], sem_s[b])
            pltpu.async_copy(td_hbm.at[dst_i.at[k]], d_bufs[b], sem_d[b])

        def g_wait(k, b):
            pltpu.make_async_copy(ts_hbm.at[src_i.at[k]], s_bufs[b],
                                  sem_s[b]).wait()
            pltpu.make_async_copy(td_hbm.at[dst_i.at[k]], d_bufs[b],
                                  sem_d[b]).wait()

        def sc_start(k, b):
            pltpu.async_copy(s_bufs[b], acc.at[dst_i.at[k]], sem_sc[b],
                             add=True)

        def sc_wait(k, b):
            pltpu.make_async_copy(s_bufs[b], acc.at[dst_i.at[k]],
                                  sem_sc[b]).wait()

        def compute(b):
            s_v = s_bufs[b]
            d_v = d_bufs[b]
            if heads == 8:
                @plsc.parallel_loop(0, BB // 2, unroll=2)
                def mbody(j):
                    e0 = 2 * j
                    e1 = 2 * j + 1
                    va0 = s_v[e0, pl.ds(64, 16)]
                    va1 = s_v[e1, pl.ds(64, 16)]
                    vd0 = d_v[e0, pl.ds(0, 16)]
                    vd1 = d_v[e1, pl.ds(0, 16)]
                    a_s = jnp.where(iota < 8, _vgather(va0, 8 + colh),
                                    _vgather(va1, 8 + colh))
                    a_d = jnp.where(iota < 8, _vgather(vd0, colh),
                                    _vgather(vd1, colh))
                    e = a_s + a_d
                    e = jnp.maximum(e, 0.2 * e)
                    w = jnp.exp(e - cvec)
                    for bb, off in ((e0, 0), (e1, 8)):
                        for t in range(5):
                            wv = _vgather(w, off + hmc[t])
                            s_v[bb, pl.ds(16 * t, 16)] = (
                                s_v[bb, pl.ds(16 * t, 16)] * wv)
            else:
                @plsc.parallel_loop(0, BB, unroll=4)
                def mbody(bb):
                    va = s_v[bb, pl.ds(64, 16)]
                    vd = d_v[bb, pl.ds(0, 16)]
                    e = _vgather(va, zero16 + 8) + _vgather(vd, zero16)
                    e = jnp.maximum(e, 0.2 * e)
                    w = jnp.exp(e - cvec)
                    for t in range(5):
                        s_v[bb, pl.ds(16 * t, 16)] = (
                            s_v[bb, pl.ds(16 * t, 16)] * w)

        g_start(0, 0)
        g_start(1, 1)
        MLAST = NBATCH // NBUF - 1

        def mloop(m, carry):
            for b in range(NBUF):
                k = NBUF * m + b
                g_wait(k, b)
                nb = (b + 2) % NBUF
                if b < 2:
                    @pl.when(m > 0)
                    def _():
                        sc_wait(k - 2, nb)
                    g_start(k + 2, nb)
                else:
                    sc_wait(k - 2, nb)

                    @pl.when(m < MLAST)
                    def _():
                        g_start(k + 2, nb)
                compute(b)
                sc_start(k, b)
            return carry

        lax.fori_loop(0, NBATCH // NBUF, mloop, 0)
        sc_wait(NBATCH - 2, 2)
        sc_wait(NBATCH - 1, 3)
        plsc.subcore_barrier()
        pltpu.sync_copy(acc.at[pl.ds(r0, RPT)],
                        out_hbm.at[pl.ds(cid * NPAD + r0, RPT)])

    return edge_pass


def _lrelu(v):
    return jnp.maximum(v, 0.2 * v)


def kernel(x, edge_index, W1, a1_src, a1_dst, b1, W2, a2_src, a2_dst, b2):
    xp = jnp.pad(x, ((0, NPAD - NN), (0, 0)))
    loops = jnp.arange(NN, dtype=edge_index.dtype)
    ei = jnp.concatenate([edge_index, jnp.stack([loops, loops])], axis=1)
    src = jnp.pad(ei[0], (0, EPAD - ETOT),
                  constant_values=NN).reshape(32 * NBATCH, BB)
    dst = jnp.pad(ei[1], (0, EPAD - ETOT),
                  constant_values=NN).reshape(32 * NBATCH, BB)
    blockmask = (lax.broadcasted_iota(jnp.int32, (64, 8), 0) // 8 ==
                 lax.broadcasted_iota(jnp.int32, (64, 8), 1))
    a1s = jnp.where(blockmask, a1_src.reshape(64, 1), 0.0)
    a1d = jnp.where(blockmask, a1_dst.reshape(64, 1), 0.0)
    a2s = jnp.pad(a2_src.reshape(64, 1), ((0, 0), (0, 7)))
    a2d = jnp.pad(a2_dst.reshape(64, 1), ((0, 0), (0, 7)))
    zrows = jnp.zeros((NPAD, WROW), _f32)

    ts1, td1, mx1 = _stage1(xp, W1, a1s, a1d)
    cv1 = jnp.tile(_lrelu(mx1[0, 0:8] + mx1[0, 8:16]), 2)
    part1 = _make_edge_pass(8)(src, dst, ts1, td1, cv1, zrows)

    embf, ts2, td2, mx2 = _stage2(part1[:NPAD], part1[NPAD:],
                                  b1.reshape(1, 64), W2, a2s, a2d)
    cv2 = jnp.full((16,), _lrelu(mx2[0, 0] + mx2[0, 8]), _f32)
    part2 = _make_edge_pass(1)(src, dst, ts2, td2, cv2, zrows)

    logp = _stage3(part2[:NPAD], part2[NPAD:], b2.reshape(1, 64))
    return (embf[:NN], logp[:NN])
